# baseline (speedup 1.0000x reference)
import functools
import math

import jax
import jax.numpy as jnp
from jax import lax
from jax.experimental import pallas as pl
from jax.experimental.pallas import tpu as pltpu
from jax.experimental.pallas import tpu_sc as plsc

N_NODES = 50000
N_PAD = 50176
N_EDGES = 3200000
NC, NS, LANES = 2, 16, 16
NW = NC * NS
EDGES_PER_W = N_EDGES // NW
CHUNK = 1000
NCHUNK = EDGES_PER_W // CHUNK
ROWS_PER_SUB = N_PAD // NS
TEMB_SCALE = math.pi / 1000.0
BE = 3200
BN = 2000
F32 = jnp.float32


def _sc_edge_body(src, dst, pre16, xw16, z16, out,
                  src_v, dst_v, gat_v, table, acc, sem):
    cid = lax.axis_index("c")
    sid = lax.axis_index("s")
    w = sid * NC + cid
    r0 = pl.multiple_of(sid * ROWS_PER_SUB, 8)
    pltpu.sync_copy(xw16.at[pl.ds(r0, ROWS_PER_SUB)],
                    table.at[pl.ds(r0, ROWS_PER_SUB)])
    pltpu.sync_copy(z16.at[pl.ds(r0, ROWS_PER_SUB)],
                    acc.at[pl.ds(r0, ROWS_PER_SUB)])
    plsc.subcore_barrier()

    zero16 = jnp.zeros((LANES,), F32)

    def chunk_body(k, _):
        base = pl.multiple_of(w * EDGES_PER_W + k * CHUNK, 8)
        pltpu.sync_copy(src.at[pl.ds(base, CHUNK)], src_v)
        pltpu.sync_copy(dst.at[pl.ds(base, CHUNK)], dst_v)
        pltpu.sync_copy(pre16.at[pl.ds(base, CHUNK)], gat_v)
        pltpu.async_copy(table.at[src_v], gat_v, sem, add=True).wait()

        def edge_body(i, _):
            gat_v[i, :] = jnp.maximum(gat_v[i, :], zero16)
            return 0

        lax.fori_loop(0, CHUNK, edge_body, 0, unroll=8)
        pltpu.sync_copy(gat_v, acc.at[dst_v], add=True)
        return 0

    lax.fori_loop(0, NCHUNK, chunk_body, 0)
    plsc.subcore_barrier()
    pltpu.sync_copy(acc.at[pl.ds(r0, ROWS_PER_SUB)],
                    out.at[cid, pl.ds(r0, ROWS_PER_SUB)])


def _sc_edge(src, dst, pre16, xw16, z16):
    fn = pl.kernel(
        _sc_edge_body,
        out_type=jax.ShapeDtypeStruct((2, N_PAD, 16), F32),
        mesh=plsc.VectorSubcoreMesh(core_axis_name="c", subcore_axis_name="s",
                                    num_cores=NC, num_subcores=NS),
        compiler_params=pltpu.CompilerParams(use_tc_tiling_on_sc=False),
        scratch_types=[
            pltpu.VMEM((CHUNK,), jnp.int32),
            pltpu.VMEM((CHUNK,), jnp.int32),
            pltpu.VMEM((CHUNK, 16), F32),
            pltpu.VMEM_SHARED((N_PAD, 16), F32),
            pltpu.VMEM_SHARED((N_PAD, 16), F32),
            pltpu.SemaphoreType.DMA,
        ],
    )
    return fn(src, dst, pre16, xw16, z16)


def _dotT(a, b):
    return lax.dot_general(a, b, (((0,), (0,)), ((), ())),
                           preferred_element_type=F32)


def _edge_prep_body(e8, eat_ref, w_ref, b_ref, o_ref):
    h = _dotT(eat_ref[...], w_ref[...]) + b_ref[...]
    one = jnp.full((BE, 1), e8, F32)
    zer = jnp.zeros((BE, 7), F32)
    o_ref[...] = jnp.concatenate([h, one, zer], axis=1)


def _edge_prep(eaT, w, b, e8):
    return pl.pallas_call(
        functools.partial(_edge_prep_body, e8),
        grid=(N_EDGES // BE,),
        in_specs=[pl.BlockSpec((7, BE), lambda i: (0, i)),
                  pl.BlockSpec((7, 8), lambda i: (0, 0)),
                  pl.BlockSpec((1, 8), lambda i: (0, 0))],
        out_specs=pl.BlockSpec((BE, 16), lambda i: (i, 0)),
        out_shape=jax.ShapeDtypeStruct((N_EDGES, 16), F32),
    )(eaT, w, b)


def _xw0_body(x_ref, w_ref, o_ref):
    o_ref[...] = jnp.dot(x_ref[...], w_ref[...], preferred_element_type=F32)


def _xw0(x0, w0p):
    return pl.pallas_call(
        _xw0_body,
        grid=(N_NODES // BN,),
        in_specs=[pl.BlockSpec((BN, 7), lambda i: (i, 0)),
                  pl.BlockSpec((7, 16), lambda i: (0, 0))],
        out_specs=pl.BlockSpec((BN, 16), lambda i: (i, 0)),
        out_shape=jax.ShapeDtypeStruct((N_PAD, 16), F32),
    )(x0, w0p)


def _node(l, Sp, S0, x, tn, conds, wl, out_dim, has_next):
    full = lambda a: pl.BlockSpec(a.shape, lambda i: (0,) * a.ndim)
    row = lambda w: pl.BlockSpec((BN, w), lambda i: (i, 0))
    p0 = pl.BlockSpec((1, BN, 16), lambda i: (0, i, 0))
    p1 = pl.BlockSpec((1, BN, 16), lambda i: (1, i, 0))
    weights = [wl['w2psi'], wl['b2psi'], wl['p1x'], wl['p1a'], wl['p1t0'],
               wl['p1t1'], wl['p1b'], wl['p2'], wl['p2b'], wl['wc'],
               wl['bc'], wl['g1'], wl['g1b'], wl['g2'], wl['g2b'],
               wl['e1'], wl['e1b'], wl['e2'], wl['e2b']]
    if has_next:
        weights.append(wl['wnext'])
    in_specs = ([p0, p1, p0, p1, row(x.shape[1]), row(1), row(4)]
                + [full(w) for w in weights])

    def body(*refs):
        sp0, sp1, c0, c1, x_r, tn_r, cond_r = refs[:7]
        wr = list(refs[7:])
        (w2psi, b2psi, p1x, p1a, p1t0, p1t1, p1b, p2, p2b, wc, bc,
         g1, g1b, g2, g2b, e1, e1b, e2, e2b) = wr[:19]
        wr = wr[19:]
        wnext = wr.pop(0) if has_next else None
        if has_next:
            out_ref, xwn_ref = wr
        else:
            (out_ref,) = wr
        relu = lambda v: jnp.maximum(v, 0.0)
        dot = functools.partial(jnp.dot, preferred_element_type=F32)

        S = sp0[0, :, 0:8] + sp1[0, :, 0:8]
        cnt = c0[0, :, 8:9] + c1[0, :, 8:9]
        inv = 1.0 / jnp.maximum(cnt, 1.0)
        agg = (dot(S, w2psi[...]) + cnt * b2psi[...]) * inv
        ce = dot(cond_r[...], wc[...]) + bc[...]
        gam = dot(relu(dot(ce, g1[...]) + g1b[...]), g2[...]) + g2b[...]
        bet = dot(relu(dot(ce, e1[...]) + e1b[...]), e2[...]) + e2b[...]
        ang = tn_r[...] * TEMB_SCALE
        u = (dot(x_r[...], p1x[...]) + dot(agg, p1a[...])
             + jnp.cos(ang) * p1t0[...] + jnp.sin(ang) * p1t1[...]
             + p1b[...])
        h = dot(relu(u), p2[...]) + p2b[...]
        o = gam * h + bet
        out_ref[...] = o
        if has_next:
            xwn_ref[...] = dot(o, wnext[...])

    out_specs = [row(out_dim)]
    out_shape = [jax.ShapeDtypeStruct((N_NODES, out_dim), F32)]
    if has_next:
        out_specs.append(row(16))
        out_shape.append(jax.ShapeDtypeStruct((N_PAD, 16), F32))
    res = pl.pallas_call(
        body,
        grid=(N_NODES // BN,),
        in_specs=in_specs,
        out_specs=out_specs,
        out_shape=out_shape,
    )(Sp, Sp, S0, S0, x, tn, conds, *weights)
    return res if has_next else (res[0], None)


_INS = (7, 8, 8)
_OUTS = (8, 8, 5)


def _prep_layer(p, in_dim, nxt_w1x):
    (w1, b1), (w2, b2) = p['psi']
    (q1, q1b), (q2, q2b) = p['phi']
    (g1, g1b), (g2, g2b) = p['gamma']
    (e1, e1b), (e2, e2b) = p['beta']
    wl = {
        'w1e': w1[in_dim:], 'b1': b1.reshape(1, -1),
        'w2psi': w2, 'b2psi': b2.reshape(1, -1),
        'p1x': q1[:in_dim], 'p1a': q1[in_dim:in_dim + 8],
        'p1t0': q1[in_dim + 8:in_dim + 9], 'p1t1': q1[in_dim + 9:in_dim + 10],
        'p1b': q1b.reshape(1, -1), 'p2': q2, 'p2b': q2b.reshape(1, -1),
        'g1': g1, 'g1b': g1b.reshape(1, -1), 'g2': g2, 'g2b': g2b.reshape(1, -1),
        'e1': e1, 'e1b': e1b.reshape(1, -1), 'e2': e2, 'e2b': e2b.reshape(1, -1),
    }
    if nxt_w1x is not None:
        wl['wnext'] = jnp.concatenate(
            [nxt_w1x, jnp.zeros_like(nxt_w1x)], axis=1)
    return wl


def kernel(x_t, active_sites, edge_index, edge_attr, conds, time_node, params):
    layers = [params['l0'], params['l1'], params['l2']]
    w1xs = [p['psi'][0][0][:din] for p, din in zip(layers, _INS)]
    wls = [_prep_layer(p, din, w1xs[i + 1] if i < 2 else None)
           for i, (p, din) in enumerate(zip(layers, _INS))]
    for wl in wls:
        wl['wc'] = params['cond'][0]
        wl['bc'] = params['cond'][1].reshape(1, -1)

    eaT = edge_attr.T
    x0 = jnp.concatenate([x_t, active_sites], axis=1)
    tn = time_node.reshape(N_NODES, 1)
    z16 = jnp.zeros((N_PAD, 16), F32)
    w0p = jnp.concatenate([w1xs[0], jnp.zeros((_INS[0], 8), F32)], axis=1)
    src = edge_index[0]
    dst = edge_index[1]

    pre0 = _edge_prep(eaT, wls[0]['w1e'], wls[0]['b1'], 1.0)
    xw16 = _xw0(x0, w0p)
    Sp0 = _sc_edge(src, dst, pre0, xw16, z16)
    pre1 = _edge_prep(eaT, wls[1]['w1e'], wls[1]['b1'], 0.0)
    x1, xw16 = _node(0, Sp0, Sp0, x0, tn, conds, wls[0], 8, True)
    Sp1 = _sc_edge(src, dst, pre1, xw16, z16)
    pre2 = _edge_prep(eaT, wls[2]['w1e'], wls[2]['b1'], 0.0)
    x2, xw16 = _node(1, Sp1, Sp0, x1, tn, conds, wls[1], 8, True)
    Sp2 = _sc_edge(src, dst, pre2, xw16, z16)
    x3, _ = _node(2, Sp2, Sp0, x2, tn, conds, wls[2], 5, False)
    return x3


# speedup vs baseline: 1.4764x; 1.4764x over previous
import functools
import math

import jax
import jax.numpy as jnp
from jax import lax
from jax.experimental import pallas as pl
from jax.experimental.pallas import tpu as pltpu
from jax.experimental.pallas import tpu_sc as plsc

N_NODES = 50000
N_PAD = 50176
N_EDGES = 3200000
NC, NS, LANES = 2, 16, 16
NW = NC * NS
EDGES_PER_W = N_EDGES // NW
CHUNK = 800
NCHUNK = EDGES_PER_W // CHUNK
ROWS_PER_SUB = N_PAD // NS
TEMB_SCALE = math.pi / 1000.0
BE = 3200
BN = 2000
F32 = jnp.float32


def _sc_edge_body(src, dst, pre16, xw16, z16, out,
                  src_v, dst_v, pre_v, gat_v, table, acc, sem):
    cid = lax.axis_index("c")
    sid = lax.axis_index("s")
    w = sid * NC + cid
    r0 = pl.multiple_of(sid * ROWS_PER_SUB, 8)
    pltpu.sync_copy(xw16.at[pl.ds(r0, ROWS_PER_SUB)],
                    table.at[pl.ds(r0, ROWS_PER_SUB)])
    pltpu.sync_copy(z16.at[pl.ds(r0, ROWS_PER_SUB)],
                    acc.at[pl.ds(r0, ROWS_PER_SUB)])
    plsc.subcore_barrier()

    zero16 = jnp.zeros((LANES,), F32)

    def chunk_body(k, _):
        base = pl.multiple_of(w * EDGES_PER_W + k * CHUNK, 8)
        pltpu.sync_copy(src.at[pl.ds(base, CHUNK)], src_v)
        pltpu.sync_copy(dst.at[pl.ds(base, CHUNK)], dst_v)
        pltpu.sync_copy(pre16.at[pl.ds(base * 16, CHUNK * 16)], pre_v)
        pltpu.async_copy(table.at[src_v], gat_v, sem).wait()

        def edge_body(i, _):
            p = pre_v[pl.ds(i * 16, 16)]
            gat_v[i, :] = jnp.maximum(gat_v[i, :] + p, zero16)
            return 0

        lax.fori_loop(0, CHUNK, edge_body, 0, unroll=8)
        pltpu.sync_copy(gat_v, acc.at[dst_v], add=True)
        return 0

    lax.fori_loop(0, NCHUNK, chunk_body, 0)
    plsc.subcore_barrier()
    pltpu.sync_copy(acc.at[pl.ds(r0, ROWS_PER_SUB)],
                    out.at[cid, pl.ds(r0, ROWS_PER_SUB)])


def _sc_edge(src, dst, pre16, xw16, z16):
    fn = pl.kernel(
        _sc_edge_body,
        out_type=jax.ShapeDtypeStruct((2, N_PAD, 16), F32),
        mesh=plsc.VectorSubcoreMesh(core_axis_name="c", subcore_axis_name="s",
                                    num_cores=NC, num_subcores=NS),
        compiler_params=pltpu.CompilerParams(use_tc_tiling_on_sc=False),
        scratch_types=[
            pltpu.VMEM((CHUNK,), jnp.int32),
            pltpu.VMEM((CHUNK,), jnp.int32),
            pltpu.VMEM((CHUNK * 16,), F32),
            pltpu.VMEM((CHUNK, 16), F32),
            pltpu.VMEM_SHARED((N_PAD, 16), F32),
            pltpu.VMEM_SHARED((N_PAD, 16), F32),
            pltpu.SemaphoreType.DMA,
        ],
    )
    return fn(src, dst, pre16, xw16, z16)


def _dotT(a, b):
    return lax.dot_general(a, b, (((0,), (0,)), ((), ())),
                           preferred_element_type=F32)


BER = 640


def _edge_prep_body(e8, *refs):
    ea = refs[:8]
    w_ref, b_ref, o_ref = refs[8:]
    one = jnp.full((BER, 1), e8, F32)
    zer = jnp.zeros((BER, 7), F32)
    parts = []
    for k in range(8):
        h = _dotT(ea[k][...], w_ref[...]) + b_ref[...]
        parts.append(jnp.concatenate([h, one, zer], axis=1))
    o_ref[...] = jnp.concatenate(parts, axis=1)


def _edge_prep(eaT, w, b, e8):
    eg = N_EDGES // 8 // BER
    in_specs = [pl.BlockSpec((7, BER), lambda i, k=k: (0, k * eg + i))
                for k in range(8)]
    in_specs += [pl.BlockSpec((7, 8), lambda i: (0, 0)),
                 pl.BlockSpec((1, 8), lambda i: (0, 0))]
    return pl.pallas_call(
        functools.partial(_edge_prep_body, e8),
        grid=(eg,),
        in_specs=in_specs,
        out_specs=pl.BlockSpec((BER, 128), lambda i: (i, 0)),
        out_shape=jax.ShapeDtypeStruct((N_EDGES // 8, 128), F32),
    )(*([eaT] * 8), w, b)


def _xw0_body(x_ref, w_ref, o_ref):
    o_ref[...] = jnp.dot(x_ref[...], w_ref[...], preferred_element_type=F32)


def _xw0(x0, w0p):
    return pl.pallas_call(
        _xw0_body,
        grid=(N_NODES // BN,),
        in_specs=[pl.BlockSpec((BN, 7), lambda i: (i, 0)),
                  pl.BlockSpec((7, 16), lambda i: (0, 0))],
        out_specs=pl.BlockSpec((BN, 16), lambda i: (i, 0)),
        out_shape=jax.ShapeDtypeStruct((N_PAD, 16), F32),
    )(x0, w0p)


def _node(l, Sp, S0, x, tn, conds, wl, out_dim, has_next):
    full = lambda a: pl.BlockSpec(a.shape, lambda i: (0,) * a.ndim)
    row = lambda w: pl.BlockSpec((BN, w), lambda i: (i, 0))
    p0 = pl.BlockSpec((1, BN, 16), lambda i: (0, i, 0))
    p1 = pl.BlockSpec((1, BN, 16), lambda i: (1, i, 0))
    weights = [wl['w2psi'], wl['b2psi'], wl['p1x'], wl['p1a'], wl['p1t0'],
               wl['p1t1'], wl['p1b'], wl['p2'], wl['p2b'], wl['wc'],
               wl['bc'], wl['g1'], wl['g1b'], wl['g2'], wl['g2b'],
               wl['e1'], wl['e1b'], wl['e2'], wl['e2b']]
    if has_next:
        weights.append(wl['wnext'])
    in_specs = ([p0, p1, p0, p1, row(x.shape[1]), row(1), row(4)]
                + [full(w) for w in weights])

    def body(*refs):
        sp0, sp1, c0, c1, x_r, tn_r, cond_r = refs[:7]
        wr = list(refs[7:])
        (w2psi, b2psi, p1x, p1a, p1t0, p1t1, p1b, p2, p2b, wc, bc,
         g1, g1b, g2, g2b, e1, e1b, e2, e2b) = wr[:19]
        wr = wr[19:]
        wnext = wr.pop(0) if has_next else None
        if has_next:
            out_ref, xwn_ref = wr
        else:
            (out_ref,) = wr
        relu = lambda v: jnp.maximum(v, 0.0)
        dot = functools.partial(jnp.dot, preferred_element_type=F32)

        S = sp0[0, :, 0:8] + sp1[0, :, 0:8]
        cnt = c0[0, :, 8:9] + c1[0, :, 8:9]
        inv = 1.0 / jnp.maximum(cnt, 1.0)
        agg = (dot(S, w2psi[...]) + cnt * b2psi[...]) * inv
        ce = dot(cond_r[...], wc[...]) + bc[...]
        gam = dot(relu(dot(ce, g1[...]) + g1b[...]), g2[...]) + g2b[...]
        bet = dot(relu(dot(ce, e1[...]) + e1b[...]), e2[...]) + e2b[...]
        ang = tn_r[...] * TEMB_SCALE
        u = (dot(x_r[...], p1x[...]) + dot(agg, p1a[...])
             + jnp.cos(ang) * p1t0[...] + jnp.sin(ang) * p1t1[...]
             + p1b[...])
        h = dot(relu(u), p2[...]) + p2b[...]
        o = gam * h + bet
        out_ref[...] = o
        if has_next:
            xwn_ref[...] = dot(o, wnext[...])

    out_specs = [row(out_dim)]
    out_shape = [jax.ShapeDtypeStruct((N_NODES, out_dim), F32)]
    if has_next:
        out_specs.append(row(16))
        out_shape.append(jax.ShapeDtypeStruct((N_PAD, 16), F32))
    res = pl.pallas_call(
        body,
        grid=(N_NODES // BN,),
        in_specs=in_specs,
        out_specs=out_specs,
        out_shape=out_shape,
    )(Sp, Sp, S0, S0, x, tn, conds, *weights)
    return res if has_next else (res[0], None)


_INS = (7, 8, 8)
_OUTS = (8, 8, 5)


def _prep_layer(p, in_dim, nxt_w1x):
    (w1, b1), (w2, b2) = p['psi']
    (q1, q1b), (q2, q2b) = p['phi']
    (g1, g1b), (g2, g2b) = p['gamma']
    (e1, e1b), (e2, e2b) = p['beta']
    wl = {
        'w1e': w1[in_dim:], 'b1': b1.reshape(1, -1),
        'w2psi': w2, 'b2psi': b2.reshape(1, -1),
        'p1x': q1[:in_dim], 'p1a': q1[in_dim:in_dim + 8],
        'p1t0': q1[in_dim + 8:in_dim + 9], 'p1t1': q1[in_dim + 9:in_dim + 10],
        'p1b': q1b.reshape(1, -1), 'p2': q2, 'p2b': q2b.reshape(1, -1),
        'g1': g1, 'g1b': g1b.reshape(1, -1), 'g2': g2, 'g2b': g2b.reshape(1, -1),
        'e1': e1, 'e1b': e1b.reshape(1, -1), 'e2': e2, 'e2b': e2b.reshape(1, -1),
    }
    if nxt_w1x is not None:
        wl['wnext'] = jnp.concatenate(
            [nxt_w1x, jnp.zeros_like(nxt_w1x)], axis=1)
    return wl


def kernel(x_t, active_sites, edge_index, edge_attr, conds, time_node, params):
    layers = [params['l0'], params['l1'], params['l2']]
    w1xs = [p['psi'][0][0][:din] for p, din in zip(layers, _INS)]
    wls = [_prep_layer(p, din, w1xs[i + 1] if i < 2 else None)
           for i, (p, din) in enumerate(zip(layers, _INS))]
    for wl in wls:
        wl['wc'] = params['cond'][0]
        wl['bc'] = params['cond'][1].reshape(1, -1)

    eaT = edge_attr.T
    x0 = jnp.concatenate([x_t, active_sites], axis=1)
    tn = time_node.reshape(N_NODES, 1)
    z16 = jnp.zeros((N_PAD, 16), F32)
    w0p = jnp.concatenate([w1xs[0], jnp.zeros((_INS[0], 8), F32)], axis=1)
    src = jnp.transpose(edge_index[0].reshape(8, N_EDGES // 8)).reshape(-1)
    dst = jnp.transpose(edge_index[1].reshape(8, N_EDGES // 8)).reshape(-1)

    pre0 = _edge_prep(eaT, wls[0]['w1e'], wls[0]['b1'], 1.0).reshape(-1)
    xw16 = _xw0(x0, w0p)
    Sp0 = _sc_edge(src, dst, pre0, xw16, z16)
    pre1 = _edge_prep(eaT, wls[1]['w1e'], wls[1]['b1'], 0.0).reshape(-1)
    x1, xw16 = _node(0, Sp0, Sp0, x0, tn, conds, wls[0], 8, True)
    Sp1 = _sc_edge(src, dst, pre1, xw16, z16)
    pre2 = _edge_prep(eaT, wls[2]['w1e'], wls[2]['b1'], 0.0).reshape(-1)
    x2, xw16 = _node(1, Sp1, Sp0, x1, tn, conds, wls[1], 8, True)
    Sp2 = _sc_edge(src, dst, pre2, xw16, z16)
    x3, _ = _node(2, Sp2, Sp0, x2, tn, conds, wls[2], 5, False)
    return x3


# speedup vs baseline: 1.8784x; 1.2722x over previous
import functools
import math

import jax
import jax.numpy as jnp
import jax.scipy.linalg
from jax import lax
from jax.experimental import pallas as pl
from jax.experimental.pallas import tpu as pltpu
from jax.experimental.pallas import tpu_sc as plsc

N_NODES = 50000
N_PAD = 50176
N_EDGES = 3200000
NC, NS, LANES = 2, 16, 16
NW = NC * NS
EDGES_PER_W = N_EDGES // NW
CHUNK = 1000
NCHUNK = EDGES_PER_W // CHUNK
ROWS_PER_SUB = N_PAD // NS
TEMB_SCALE = math.pi / 1000.0
EA_FEATS = 7
BE = 3200
BN = 2000
F32 = jnp.float32


def _sc_edge_body(sd, pre16, xw16, z16, out,
                  sd0, sd1, pre0, pre1, gat0, gat1, acc,
                  sem0, sem1, gsem):
    cid = lax.axis_index("c")
    sid = lax.axis_index("s")
    w = sid * NC + cid
    r0 = pl.multiple_of(sid * ROWS_PER_SUB, 8)
    pltpu.sync_copy(z16.at[pl.ds(r0, ROWS_PER_SUB)],
                    acc.at[pl.ds(r0, ROWS_PER_SUB)])
    plsc.subcore_barrier()

    zero16 = jnp.zeros((LANES,), F32)
    bufs = ((sd0, pre0, gat0, sem0), (sd1, pre1, gat1, sem1))

    def fire(c, b):
        sd_v, pre_v, _, sem = bufs[b]
        base = pl.multiple_of(w * EDGES_PER_W + c * CHUNK, 8)
        pltpu.async_copy(sd.at[:, pl.ds(base, CHUNK)], sd_v, sem)
        pltpu.async_copy(pre16.at[pl.ds(base * 16, CHUNK * 16)], pre_v, sem)

    def process(b):
        sd_v, pre_v, gat_v, sem = bufs[b]
        pltpu.make_async_copy(sd.at[:, pl.ds(0, CHUNK)], sd_v, sem).wait()
        pltpu.make_async_copy(pre16.at[pl.ds(0, CHUNK * 16)], pre_v,
                              sem).wait()
        pltpu.async_copy(xw16.at[sd_v.at[0]], gat_v, gsem).wait()

        def edge_body(i, _):
            p = pre_v[pl.ds(i * 16, 16)]
            gat_v[i, :] = jnp.maximum(gat_v[i, :] + p, zero16)
            return 0

        lax.fori_loop(0, CHUNK, edge_body, 0, unroll=8)
        pltpu.sync_copy(gat_v, acc.at[sd_v.at[1]], add=True)

    fire(0, 0)

    def body2(m, _):
        fire(2 * m + 1, 1)
        process(0)

        @pl.when(m + 1 < NCHUNK // 2)
        def _():
            fire(2 * m + 2, 0)

        process(1)
        return 0

    lax.fori_loop(0, NCHUNK // 2, body2, 0)
    plsc.subcore_barrier()
    pltpu.sync_copy(acc.at[pl.ds(r0, ROWS_PER_SUB)],
                    out.at[cid, pl.ds(r0, ROWS_PER_SUB)])


def _sc_edge(sd, pre16, xw16, z16):
    fn = pl.kernel(
        _sc_edge_body,
        out_type=jax.ShapeDtypeStruct((2, N_PAD, 16), F32),
        mesh=plsc.VectorSubcoreMesh(core_axis_name="c", subcore_axis_name="s",
                                    num_cores=NC, num_subcores=NS),
        compiler_params=pltpu.CompilerParams(use_tc_tiling_on_sc=False),
        scratch_types=[
            pltpu.VMEM((2, CHUNK), jnp.int32),
            pltpu.VMEM((2, CHUNK), jnp.int32),
            pltpu.VMEM((CHUNK * 16,), F32),
            pltpu.VMEM((CHUNK * 16,), F32),
            pltpu.VMEM((CHUNK, 16), F32),
            pltpu.VMEM((CHUNK, 16), F32),
            pltpu.VMEM_SHARED((N_PAD, 16), F32),
            pltpu.SemaphoreType.DMA,
            pltpu.SemaphoreType.DMA,
            pltpu.SemaphoreType.DMA,
        ],
    )
    return fn(sd, pre16, xw16, z16)


def _dotT(a, b):
    return lax.dot_general(a, b, (((0,), (0,)), ((), ())),
                           preferred_element_type=F32)


BER = 640


def _edge_prep_body(*refs):
    ea = refs[:8]
    wblk_ref, b128_ref, o_ref = refs[8:]
    cat = jnp.concatenate([r[...] for r in ea], axis=0)
    o_ref[...] = _dotT(cat, wblk_ref[...]) + b128_ref[...]


def _edge_prep(eaT, wblk, b128):
    eg = N_EDGES // 8 // BER
    in_specs = [pl.BlockSpec((7, BER), lambda i, k=k: (0, k * eg + i))
                for k in range(8)]
    in_specs += [pl.BlockSpec((56, 128), lambda i: (0, 0)),
                 pl.BlockSpec((1, 128), lambda i: (0, 0))]
    return pl.pallas_call(
        _edge_prep_body,
        grid=(eg,),
        in_specs=in_specs,
        out_specs=pl.BlockSpec((BER, 128), lambda i: (i, 0)),
        out_shape=jax.ShapeDtypeStruct((N_EDGES // 8, 128), F32),
    )(*([eaT] * 8), wblk, b128)


def _xw0_body(x_ref, w_ref, o_ref):
    o_ref[...] = jnp.dot(x_ref[...], w_ref[...], preferred_element_type=F32)


def _xw0(x0, w0p):
    return pl.pallas_call(
        _xw0_body,
        grid=(N_NODES // BN,),
        in_specs=[pl.BlockSpec((BN, 7), lambda i: (i, 0)),
                  pl.BlockSpec((7, 16), lambda i: (0, 0))],
        out_specs=pl.BlockSpec((BN, 16), lambda i: (i, 0)),
        out_shape=jax.ShapeDtypeStruct((N_PAD, 16), F32),
    )(x0, w0p)


def _node(l, Sp, S0, x, tn, conds, wl, out_dim, has_next):
    full = lambda a: pl.BlockSpec(a.shape, lambda i: (0,) * a.ndim)
    row = lambda w: pl.BlockSpec((BN, w), lambda i: (i, 0))
    p0 = pl.BlockSpec((1, BN, 16), lambda i: (0, i, 0))
    p1 = pl.BlockSpec((1, BN, 16), lambda i: (1, i, 0))
    weights = [wl['w2psi'], wl['b2psi'], wl['p1x'], wl['p1a'], wl['p1t0'],
               wl['p1t1'], wl['p1b'], wl['p2'], wl['p2b'], wl['wc'],
               wl['bc'], wl['g1'], wl['g1b'], wl['g2'], wl['g2b'],
               wl['e1'], wl['e1b'], wl['e2'], wl['e2b']]
    if has_next:
        weights.append(wl['wnext'])
    in_specs = ([p0, p1, p0, p1, row(x.shape[1]), row(1), row(4)]
                + [full(w) for w in weights])

    def body(*refs):
        sp0, sp1, c0, c1, x_r, tn_r, cond_r = refs[:7]
        wr = list(refs[7:])
        (w2psi, b2psi, p1x, p1a, p1t0, p1t1, p1b, p2, p2b, wc, bc,
         g1, g1b, g2, g2b, e1, e1b, e2, e2b) = wr[:19]
        wr = wr[19:]
        wnext = wr.pop(0) if has_next else None
        if has_next:
            out_ref, xwn_ref = wr
        else:
            (out_ref,) = wr
        relu = lambda v: jnp.maximum(v, 0.0)
        dot = functools.partial(jnp.dot, preferred_element_type=F32)

        S = sp0[0, :, 0:8] + sp1[0, :, 0:8]
        cnt = c0[0, :, 8:9] + c1[0, :, 8:9]
        inv = 1.0 / jnp.maximum(cnt, 1.0)
        agg = (dot(S, w2psi[...]) + cnt * b2psi[...]) * inv
        ce = dot(cond_r[...], wc[...]) + bc[...]
        gam = dot(relu(dot(ce, g1[...]) + g1b[...]), g2[...]) + g2b[...]
        bet = dot(relu(dot(ce, e1[...]) + e1b[...]), e2[...]) + e2b[...]
        ang = tn_r[...] * TEMB_SCALE
        u = (dot(x_r[...], p1x[...]) + dot(agg, p1a[...])
             + jnp.cos(ang) * p1t0[...] + jnp.sin(ang) * p1t1[...]
             + p1b[...])
        h = dot(relu(u), p2[...]) + p2b[...]
        o = gam * h + bet
        out_ref[...] = o
        if has_next:
            xwn_ref[...] = dot(o, wnext[...])

    out_specs = [row(out_dim)]
    out_shape = [jax.ShapeDtypeStruct((N_NODES, out_dim), F32)]
    if has_next:
        out_specs.append(row(16))
        out_shape.append(jax.ShapeDtypeStruct((N_PAD, 16), F32))
    res = pl.pallas_call(
        body,
        grid=(N_NODES // BN,),
        in_specs=in_specs,
        out_specs=out_specs,
        out_shape=out_shape,
    )(Sp, Sp, S0, S0, x, tn, conds, *weights)
    return res if has_next else (res[0], None)


_INS = (7, 8, 8)
_OUTS = (8, 8, 5)


def _prep_layer(p, in_dim, nxt_w1x):
    (w1, b1), (w2, b2) = p['psi']
    (q1, q1b), (q2, q2b) = p['phi']
    (g1, g1b), (g2, g2b) = p['gamma']
    (e1, e1b), (e2, e2b) = p['beta']
    wl = {
        'w1e': w1[in_dim:], 'b1': b1.reshape(1, -1),
        'w2psi': w2, 'b2psi': b2.reshape(1, -1),
        'p1x': q1[:in_dim], 'p1a': q1[in_dim:in_dim + 8],
        'p1t0': q1[in_dim + 8:in_dim + 9], 'p1t1': q1[in_dim + 9:in_dim + 10],
        'p1b': q1b.reshape(1, -1), 'p2': q2, 'p2b': q2b.reshape(1, -1),
        'g1': g1, 'g1b': g1b.reshape(1, -1), 'g2': g2, 'g2b': g2b.reshape(1, -1),
        'e1': e1, 'e1b': e1b.reshape(1, -1), 'e2': e2, 'e2b': e2b.reshape(1, -1),
    }
    if nxt_w1x is not None:
        wl['wnext'] = jnp.concatenate(
            [nxt_w1x, jnp.zeros_like(nxt_w1x)], axis=1)
    return wl


def kernel(x_t, active_sites, edge_index, edge_attr, conds, time_node, params):
    layers = [params['l0'], params['l1'], params['l2']]
    w1xs = [p['psi'][0][0][:din] for p, din in zip(layers, _INS)]
    wls = [_prep_layer(p, din, w1xs[i + 1] if i < 2 else None)
           for i, (p, din) in enumerate(zip(layers, _INS))]
    for wl in wls:
        wl['wc'] = params['cond'][0]
        wl['bc'] = params['cond'][1].reshape(1, -1)

    eaT = edge_attr.T
    x0 = jnp.concatenate([x_t, active_sites], axis=1)
    tn = time_node.reshape(N_NODES, 1)
    z16 = jnp.zeros((N_PAD, 16), F32)
    w0p = jnp.concatenate([w1xs[0], jnp.zeros((_INS[0], 8), F32)], axis=1)
    sd = jnp.transpose(
        edge_index.reshape(2, 8, N_EDGES // 8), (0, 2, 1)).reshape(2, -1)

    def wblk(l, e8):
        wpad = jnp.concatenate(
            [wls[l]['w1e'],
             jnp.zeros((EA_FEATS, 8), F32)], axis=1)
        blk = jax.scipy.linalg.block_diag(*([wpad] * 8))
        brow = jnp.concatenate(
            [wls[l]['b1'], jnp.full((1, 1), e8, F32),
             jnp.zeros((1, 7), F32)], axis=1)
        return blk, jnp.tile(brow, (1, 8))

    wb0, bb0 = wblk(0, 1.0)
    wb1, bb1 = wblk(1, 0.0)
    wb2, bb2 = wblk(2, 0.0)

    pre0 = _edge_prep(eaT, wb0, bb0).reshape(-1)
    xw16 = _xw0(x0, w0p)
    Sp0 = _sc_edge(sd, pre0, xw16, z16)
    pre1 = _edge_prep(eaT, wb1, bb1).reshape(-1)
    x1, xw16 = _node(0, Sp0, Sp0, x0, tn, conds, wls[0], 8, True)
    Sp1 = _sc_edge(sd, pre1, xw16, z16)
    pre2 = _edge_prep(eaT, wb2, bb2).reshape(-1)
    x2, xw16 = _node(1, Sp1, Sp0, x1, tn, conds, wls[1], 8, True)
    Sp2 = _sc_edge(sd, pre2, xw16, z16)
    x3, _ = _node(2, Sp2, Sp0, x2, tn, conds, wls[2], 5, False)
    return x3


# speedup vs baseline: 2.0399x; 1.0860x over previous
import functools
import math

import jax
import jax.numpy as jnp
import jax.scipy.linalg
from jax import lax
from jax.experimental import pallas as pl
from jax.experimental.pallas import tpu as pltpu
from jax.experimental.pallas import tpu_sc as plsc

N_NODES = 50000
N_PAD = 50176
N_EDGES = 3200000
NC, NS, LANES = 2, 16, 16
NW = NC * NS
EDGES_PER_W = N_EDGES // NW
CHUNK = 400
NCHUNK = EDGES_PER_W // CHUNK
ROWS_PER_SUB = N_PAD // NS
TEMB_SCALE = math.pi / 1000.0
EA_FEATS = 7
BE = 3200
BN = 2000
F32 = jnp.float32


def _sc_edge_body(sd, pre16, xw16, z16, out,
                  sd0, sd1, pre0, pre1, gat0, gat1, table, acc,
                  sem0, sem1, gsem):
    cid = lax.axis_index("c")
    sid = lax.axis_index("s")
    w = sid * NC + cid
    r0 = pl.multiple_of(sid * ROWS_PER_SUB, 8)
    pltpu.sync_copy(xw16.at[pl.ds(r0, ROWS_PER_SUB)],
                    table.at[pl.ds(r0, ROWS_PER_SUB)])
    pltpu.sync_copy(z16.at[pl.ds(r0, ROWS_PER_SUB)],
                    acc.at[pl.ds(r0, ROWS_PER_SUB)])
    plsc.subcore_barrier()

    zero16 = jnp.zeros((LANES,), F32)
    bufs = ((sd0, pre0, gat0, sem0), (sd1, pre1, gat1, sem1))

    def fire(c, b):
        sd_v, pre_v, _, sem = bufs[b]
        base = pl.multiple_of(w * EDGES_PER_W + c * CHUNK, 8)
        pltpu.async_copy(sd.at[:, pl.ds(base, CHUNK)], sd_v, sem)
        pltpu.async_copy(pre16.at[pl.ds(base * 16, CHUNK * 16)], pre_v, sem)

    def process(b):
        sd_v, pre_v, gat_v, sem = bufs[b]
        pltpu.make_async_copy(sd.at[:, pl.ds(0, CHUNK)], sd_v, sem).wait()
        pltpu.make_async_copy(pre16.at[pl.ds(0, CHUNK * 16)], pre_v,
                              sem).wait()
        pltpu.async_copy(table.at[sd_v.at[0]], gat_v, gsem).wait()

        def edge_body(i, _):
            p = pre_v[pl.ds(i * 16, 16)]
            gat_v[i, :] = jnp.maximum(gat_v[i, :] + p, zero16)
            return 0

        lax.fori_loop(0, CHUNK, edge_body, 0, unroll=8)
        pltpu.sync_copy(gat_v, acc.at[sd_v.at[1]], add=True)

    fire(0, 0)

    def body2(m, _):
        fire(2 * m + 1, 1)
        process(0)

        @pl.when(m + 1 < NCHUNK // 2)
        def _():
            fire(2 * m + 2, 0)

        process(1)
        return 0

    lax.fori_loop(0, NCHUNK // 2, body2, 0)
    plsc.subcore_barrier()
    pltpu.sync_copy(acc.at[pl.ds(r0, ROWS_PER_SUB)],
                    out.at[cid, pl.ds(r0, ROWS_PER_SUB)])


def _sc_edge(sd, pre16, xw16, z16):
    fn = pl.kernel(
        _sc_edge_body,
        out_type=jax.ShapeDtypeStruct((2, N_PAD, 16), F32),
        mesh=plsc.VectorSubcoreMesh(core_axis_name="c", subcore_axis_name="s",
                                    num_cores=NC, num_subcores=NS),
        compiler_params=pltpu.CompilerParams(use_tc_tiling_on_sc=False),
        scratch_types=[
            pltpu.VMEM((2, CHUNK), jnp.int32),
            pltpu.VMEM((2, CHUNK), jnp.int32),
            pltpu.VMEM((CHUNK * 16,), F32),
            pltpu.VMEM((CHUNK * 16,), F32),
            pltpu.VMEM((CHUNK, 16), F32),
            pltpu.VMEM((CHUNK, 16), F32),
            pltpu.VMEM_SHARED((N_PAD, 16), F32),
            pltpu.VMEM_SHARED((N_PAD, 16), F32),
            pltpu.SemaphoreType.DMA,
            pltpu.SemaphoreType.DMA,
            pltpu.SemaphoreType.DMA,
        ],
    )
    return fn(sd, pre16, xw16, z16)


def _dotT(a, b):
    return lax.dot_general(a, b, (((0,), (0,)), ((), ())),
                           preferred_element_type=F32)


BER = 640


def _edge_prep_body(*refs):
    ea = refs[:8]
    wblk_ref, b128_ref, o_ref = refs[8:]
    cat = jnp.concatenate([r[...] for r in ea], axis=0)
    o_ref[...] = _dotT(cat, wblk_ref[...]) + b128_ref[...]


def _edge_prep(eaT, wblk, b128):
    eg = N_EDGES // 8 // BER
    in_specs = [pl.BlockSpec((7, BER), lambda i, k=k: (0, k * eg + i))
                for k in range(8)]
    in_specs += [pl.BlockSpec((56, 128), lambda i: (0, 0)),
                 pl.BlockSpec((1, 128), lambda i: (0, 0))]
    return pl.pallas_call(
        _edge_prep_body,
        grid=(eg,),
        in_specs=in_specs,
        out_specs=pl.BlockSpec((BER, 128), lambda i: (i, 0)),
        out_shape=jax.ShapeDtypeStruct((N_EDGES // 8, 128), F32),
    )(*([eaT] * 8), wblk, b128)


def _xw0_body(x_ref, w_ref, o_ref):
    o_ref[...] = jnp.dot(x_ref[...], w_ref[...], preferred_element_type=F32)


def _xw0(x0, w0p):
    return pl.pallas_call(
        _xw0_body,
        grid=(N_NODES // BN,),
        in_specs=[pl.BlockSpec((BN, 7), lambda i: (i, 0)),
                  pl.BlockSpec((7, 16), lambda i: (0, 0))],
        out_specs=pl.BlockSpec((BN, 16), lambda i: (i, 0)),
        out_shape=jax.ShapeDtypeStruct((N_PAD, 16), F32),
    )(x0, w0p)


def _node(l, Sp, S0, x, tn, conds, wl, out_dim, has_next):
    full = lambda a: pl.BlockSpec(a.shape, lambda i: (0,) * a.ndim)
    row = lambda w: pl.BlockSpec((BN, w), lambda i: (i, 0))
    p0 = pl.BlockSpec((1, BN, 16), lambda i: (0, i, 0))
    p1 = pl.BlockSpec((1, BN, 16), lambda i: (1, i, 0))
    weights = [wl['w2psi'], wl['b2psi'], wl['p1x'], wl['p1a'], wl['p1t0'],
               wl['p1t1'], wl['p1b'], wl['p2'], wl['p2b'], wl['wc'],
               wl['bc'], wl['g1'], wl['g1b'], wl['g2'], wl['g2b'],
               wl['e1'], wl['e1b'], wl['e2'], wl['e2b']]
    if has_next:
        weights.append(wl['wnext'])
    in_specs = ([p0, p1, p0, p1, row(x.shape[1]), row(1), row(4)]
                + [full(w) for w in weights])

    def body(*refs):
        sp0, sp1, c0, c1, x_r, tn_r, cond_r = refs[:7]
        wr = list(refs[7:])
        (w2psi, b2psi, p1x, p1a, p1t0, p1t1, p1b, p2, p2b, wc, bc,
         g1, g1b, g2, g2b, e1, e1b, e2, e2b) = wr[:19]
        wr = wr[19:]
        wnext = wr.pop(0) if has_next else None
        if has_next:
            out_ref, xwn_ref = wr
        else:
            (out_ref,) = wr
        relu = lambda v: jnp.maximum(v, 0.0)
        dot = functools.partial(jnp.dot, preferred_element_type=F32)

        S = sp0[0, :, 0:8] + sp1[0, :, 0:8]
        cnt = c0[0, :, 8:9] + c1[0, :, 8:9]
        inv = 1.0 / jnp.maximum(cnt, 1.0)
        agg = (dot(S, w2psi[...]) + cnt * b2psi[...]) * inv
        ce = dot(cond_r[...], wc[...]) + bc[...]
        gam = dot(relu(dot(ce, g1[...]) + g1b[...]), g2[...]) + g2b[...]
        bet = dot(relu(dot(ce, e1[...]) + e1b[...]), e2[...]) + e2b[...]
        ang = tn_r[...] * TEMB_SCALE
        u = (dot(x_r[...], p1x[...]) + dot(agg, p1a[...])
             + jnp.cos(ang) * p1t0[...] + jnp.sin(ang) * p1t1[...]
             + p1b[...])
        h = dot(relu(u), p2[...]) + p2b[...]
        o = gam * h + bet
        out_ref[...] = o
        if has_next:
            xwn_ref[...] = dot(o, wnext[...])

    out_specs = [row(out_dim)]
    out_shape = [jax.ShapeDtypeStruct((N_NODES, out_dim), F32)]
    if has_next:
        out_specs.append(row(16))
        out_shape.append(jax.ShapeDtypeStruct((N_PAD, 16), F32))
    res = pl.pallas_call(
        body,
        grid=(N_NODES // BN,),
        in_specs=in_specs,
        out_specs=out_specs,
        out_shape=out_shape,
    )(Sp, Sp, S0, S0, x, tn, conds, *weights)
    return res if has_next else (res[0], None)


_INS = (7, 8, 8)
_OUTS = (8, 8, 5)


def _prep_layer(p, in_dim, nxt_w1x):
    (w1, b1), (w2, b2) = p['psi']
    (q1, q1b), (q2, q2b) = p['phi']
    (g1, g1b), (g2, g2b) = p['gamma']
    (e1, e1b), (e2, e2b) = p['beta']
    wl = {
        'w1e': w1[in_dim:], 'b1': b1.reshape(1, -1),
        'w2psi': w2, 'b2psi': b2.reshape(1, -1),
        'p1x': q1[:in_dim], 'p1a': q1[in_dim:in_dim + 8],
        'p1t0': q1[in_dim + 8:in_dim + 9], 'p1t1': q1[in_dim + 9:in_dim + 10],
        'p1b': q1b.reshape(1, -1), 'p2': q2, 'p2b': q2b.reshape(1, -1),
        'g1': g1, 'g1b': g1b.reshape(1, -1), 'g2': g2, 'g2b': g2b.reshape(1, -1),
        'e1': e1, 'e1b': e1b.reshape(1, -1), 'e2': e2, 'e2b': e2b.reshape(1, -1),
    }
    if nxt_w1x is not None:
        wl['wnext'] = jnp.concatenate(
            [nxt_w1x, jnp.zeros_like(nxt_w1x)], axis=1)
    return wl


def kernel(x_t, active_sites, edge_index, edge_attr, conds, time_node, params):
    layers = [params['l0'], params['l1'], params['l2']]
    w1xs = [p['psi'][0][0][:din] for p, din in zip(layers, _INS)]
    wls = [_prep_layer(p, din, w1xs[i + 1] if i < 2 else None)
           for i, (p, din) in enumerate(zip(layers, _INS))]
    for wl in wls:
        wl['wc'] = params['cond'][0]
        wl['bc'] = params['cond'][1].reshape(1, -1)

    eaT = edge_attr.T
    x0 = jnp.concatenate([x_t, active_sites], axis=1)
    tn = time_node.reshape(N_NODES, 1)
    z16 = jnp.zeros((N_PAD, 16), F32)
    w0p = jnp.concatenate([w1xs[0], jnp.zeros((_INS[0], 8), F32)], axis=1)
    sd = jnp.transpose(
        edge_index.reshape(2, 8, N_EDGES // 8), (0, 2, 1)).reshape(2, -1)

    def wblk(l, e8):
        wpad = jnp.concatenate(
            [wls[l]['w1e'],
             jnp.zeros((EA_FEATS, 8), F32)], axis=1)
        blk = jax.scipy.linalg.block_diag(*([wpad] * 8))
        brow = jnp.concatenate(
            [wls[l]['b1'], jnp.full((1, 1), e8, F32),
             jnp.zeros((1, 7), F32)], axis=1)
        return blk, jnp.tile(brow, (1, 8))

    wb0, bb0 = wblk(0, 1.0)
    wb1, bb1 = wblk(1, 0.0)
    wb2, bb2 = wblk(2, 0.0)

    pre0 = _edge_prep(eaT, wb0, bb0).reshape(-1)
    xw16 = _xw0(x0, w0p)
    Sp0 = _sc_edge(sd, pre0, xw16, z16)
    pre1 = _edge_prep(eaT, wb1, bb1).reshape(-1)
    x1, xw16 = _node(0, Sp0, Sp0, x0, tn, conds, wls[0], 8, True)
    Sp1 = _sc_edge(sd, pre1, xw16, z16)
    pre2 = _edge_prep(eaT, wb2, bb2).reshape(-1)
    x2, xw16 = _node(1, Sp1, Sp0, x1, tn, conds, wls[1], 8, True)
    Sp2 = _sc_edge(sd, pre2, xw16, z16)
    x3, _ = _node(2, Sp2, Sp0, x2, tn, conds, wls[2], 5, False)
    return x3


# speedup vs baseline: 2.1747x; 1.0661x over previous
import functools
import math

import jax
import jax.numpy as jnp
import jax.scipy.linalg
from jax import lax
from jax.experimental import pallas as pl
from jax.experimental.pallas import tpu as pltpu
from jax.experimental.pallas import tpu_sc as plsc

N_NODES = 50000
N_PAD = 50176
N_EDGES = 3200000
NC, NS, LANES = 2, 16, 16
NW = NC * NS
EDGES_PER_W = N_EDGES // NW
CHUNK = 400
NCHUNK = EDGES_PER_W // CHUNK
ROWS_PER_SUB = N_PAD // NS
TEMB_SCALE = math.pi / 1000.0
EA_FEATS = 7
BE = 3200
BN = 2000
F32 = jnp.float32


def _sc_edge_body(sd, pre16, xw16, z16, out,
                  sd0, sd1, pre0, pre1, gat0, gat1, dsc0, dsc1, table, acc,
                  sem0, sem1, gsem0, gsem1, ssem0, ssem1):
    cid = lax.axis_index("c")
    sid = lax.axis_index("s")
    w = sid * NC + cid
    r0 = pl.multiple_of(sid * ROWS_PER_SUB, 8)
    pltpu.sync_copy(xw16.at[pl.ds(r0, ROWS_PER_SUB)],
                    table.at[pl.ds(r0, ROWS_PER_SUB)])
    pltpu.sync_copy(z16.at[pl.ds(r0, ROWS_PER_SUB)],
                    acc.at[pl.ds(r0, ROWS_PER_SUB)])
    plsc.subcore_barrier()

    zero16 = jnp.zeros((LANES,), F32)
    bufs = ((sd0, pre0, gat0, dsc0, sem0, gsem0, ssem0),
            (sd1, pre1, gat1, dsc1, sem1, gsem1, ssem1))

    def fire_idx(c, b):
        sd_v, pre_v = bufs[b][0], bufs[b][1]
        sem = bufs[b][4]
        base = pl.multiple_of(w * EDGES_PER_W + c * CHUNK, 8)
        pltpu.async_copy(sd.at[:, pl.ds(base, CHUNK)], sd_v, sem)
        pltpu.async_copy(pre16.at[pl.ds(base * 16, CHUNK * 16)], pre_v, sem)

    def drain_idx(b):
        sd_v, pre_v = bufs[b][0], bufs[b][1]
        sem = bufs[b][4]
        pltpu.make_async_copy(sd.at[:, pl.ds(0, CHUNK)], sd_v, sem).wait()
        pltpu.make_async_copy(pre16.at[pl.ds(0, CHUNK * 16)], pre_v,
                              sem).wait()

    def start_gather(b):
        sd_v, gat_v, gsem = bufs[b][0], bufs[b][2], bufs[b][5]
        pltpu.async_copy(table.at[sd_v.at[0]], gat_v, gsem)

    def wait_gather(b):
        sd_v, gat_v, gsem = bufs[b][0], bufs[b][2], bufs[b][5]
        pltpu.make_async_copy(table.at[sd_v.at[0]], gat_v, gsem).wait()

    def start_scatter(b):
        gat_v, dsc_v, ssem = bufs[b][2], bufs[b][3], bufs[b][6]
        pltpu.async_copy(gat_v, acc.at[dsc_v], ssem, add=True)

    def wait_scatter(b):
        gat_v, dsc_v, ssem = bufs[b][2], bufs[b][3], bufs[b][6]
        pltpu.make_async_copy(gat_v, acc.at[dsc_v], ssem).wait()

    def step(c, b):
        sd_v, pre_v, gat_v, dsc_v = bufs[b][:4]
        wait_gather(b)

        def cp(j, _):
            dsc_v[pl.ds(j * 16, 16)] = sd_v[1, pl.ds(j * 16, 16)]
            return 0

        lax.fori_loop(0, CHUNK // 16, cp, 0, unroll=4)

        @pl.when(c + 2 < NCHUNK)
        def _():
            fire_idx(c + 2, b)

        @pl.when(c >= 1)
        def _():
            wait_scatter(1 - b)

        @pl.when(c + 1 < NCHUNK)
        def _():
            drain_idx(1 - b)
            start_gather(1 - b)

        def edge_body(i, _):
            p = pre_v[pl.ds(i * 16, 16)]
            gat_v[i, :] = jnp.maximum(gat_v[i, :] + p, zero16)
            return 0

        lax.fori_loop(0, CHUNK, edge_body, 0, unroll=8)
        start_scatter(b)

    fire_idx(0, 0)
    fire_idx(1, 1)
    drain_idx(0)
    start_gather(0)

    def body2(m, _):
        step(2 * m, 0)
        step(2 * m + 1, 1)
        return 0

    lax.fori_loop(0, NCHUNK // 2, body2, 0)
    wait_scatter(1)
    plsc.subcore_barrier()
    pltpu.sync_copy(acc.at[pl.ds(r0, ROWS_PER_SUB)],
                    out.at[cid, pl.ds(r0, ROWS_PER_SUB)])


def _sc_edge(sd, pre16, xw16, z16):
    fn = pl.kernel(
        _sc_edge_body,
        out_type=jax.ShapeDtypeStruct((2, N_PAD, 16), F32),
        mesh=plsc.VectorSubcoreMesh(core_axis_name="c", subcore_axis_name="s",
                                    num_cores=NC, num_subcores=NS),
        compiler_params=pltpu.CompilerParams(use_tc_tiling_on_sc=False),
        scratch_types=[
            pltpu.VMEM((2, CHUNK), jnp.int32),
            pltpu.VMEM((2, CHUNK), jnp.int32),
            pltpu.VMEM((CHUNK * 16,), F32),
            pltpu.VMEM((CHUNK * 16,), F32),
            pltpu.VMEM((CHUNK, 16), F32),
            pltpu.VMEM((CHUNK, 16), F32),
            pltpu.VMEM((CHUNK,), jnp.int32),
            pltpu.VMEM((CHUNK,), jnp.int32),
            pltpu.VMEM_SHARED((N_PAD, 16), F32),
            pltpu.VMEM_SHARED((N_PAD, 16), F32),
            pltpu.SemaphoreType.DMA,
            pltpu.SemaphoreType.DMA,
            pltpu.SemaphoreType.DMA,
            pltpu.SemaphoreType.DMA,
            pltpu.SemaphoreType.DMA,
            pltpu.SemaphoreType.DMA,
        ],
    )
    return fn(sd, pre16, xw16, z16)


def _dotT(a, b):
    return lax.dot_general(a, b, (((0,), (0,)), ((), ())),
                           preferred_element_type=F32)


BER = 640


def _edge_prep_body(*refs):
    ea = refs[:8]
    wblk_ref, b128_ref, o_ref = refs[8:]
    cat = jnp.concatenate([r[...] for r in ea], axis=0)
    o_ref[...] = _dotT(cat, wblk_ref[...]) + b128_ref[...]


def _edge_prep(eaT, wblk, b128):
    eg = N_EDGES // 8 // BER
    in_specs = [pl.BlockSpec((7, BER), lambda i, k=k: (0, k * eg + i))
                for k in range(8)]
    in_specs += [pl.BlockSpec((56, 128), lambda i: (0, 0)),
                 pl.BlockSpec((1, 128), lambda i: (0, 0))]
    return pl.pallas_call(
        _edge_prep_body,
        grid=(eg,),
        in_specs=in_specs,
        out_specs=pl.BlockSpec((BER, 128), lambda i: (i, 0)),
        out_shape=jax.ShapeDtypeStruct((N_EDGES // 8, 128), F32),
    )(*([eaT] * 8), wblk, b128)


def _xw0_body(x_ref, w_ref, o_ref):
    o_ref[...] = jnp.dot(x_ref[...], w_ref[...], preferred_element_type=F32)


def _xw0(x0, w0p):
    return pl.pallas_call(
        _xw0_body,
        grid=(N_NODES // BN,),
        in_specs=[pl.BlockSpec((BN, 7), lambda i: (i, 0)),
                  pl.BlockSpec((7, 16), lambda i: (0, 0))],
        out_specs=pl.BlockSpec((BN, 16), lambda i: (i, 0)),
        out_shape=jax.ShapeDtypeStruct((N_PAD, 16), F32),
    )(x0, w0p)


def _node(l, Sp, S0, x, tn, conds, wl, out_dim, has_next):
    full = lambda a: pl.BlockSpec(a.shape, lambda i: (0,) * a.ndim)
    row = lambda w: pl.BlockSpec((BN, w), lambda i: (i, 0))
    p0 = pl.BlockSpec((1, BN, 16), lambda i: (0, i, 0))
    p1 = pl.BlockSpec((1, BN, 16), lambda i: (1, i, 0))
    weights = [wl['w2psi'], wl['b2psi'], wl['p1x'], wl['p1a'], wl['p1t0'],
               wl['p1t1'], wl['p1b'], wl['p2'], wl['p2b'], wl['wc'],
               wl['bc'], wl['g1'], wl['g1b'], wl['g2'], wl['g2b'],
               wl['e1'], wl['e1b'], wl['e2'], wl['e2b']]
    if has_next:
        weights.append(wl['wnext'])
    in_specs = ([p0, p1, p0, p1, row(x.shape[1]), row(1), row(4)]
                + [full(w) for w in weights])

    def body(*refs):
        sp0, sp1, c0, c1, x_r, tn_r, cond_r = refs[:7]
        wr = list(refs[7:])
        (w2psi, b2psi, p1x, p1a, p1t0, p1t1, p1b, p2, p2b, wc, bc,
         g1, g1b, g2, g2b, e1, e1b, e2, e2b) = wr[:19]
        wr = wr[19:]
        wnext = wr.pop(0) if has_next else None
        if has_next:
            out_ref, xwn_ref = wr
        else:
            (out_ref,) = wr
        relu = lambda v: jnp.maximum(v, 0.0)
        dot = functools.partial(jnp.dot, preferred_element_type=F32)

        S = sp0[0, :, 0:8] + sp1[0, :, 0:8]
        cnt = c0[0, :, 8:9] + c1[0, :, 8:9]
        inv = 1.0 / jnp.maximum(cnt, 1.0)
        agg = (dot(S, w2psi[...]) + cnt * b2psi[...]) * inv
        ce = dot(cond_r[...], wc[...]) + bc[...]
        gam = dot(relu(dot(ce, g1[...]) + g1b[...]), g2[...]) + g2b[...]
        bet = dot(relu(dot(ce, e1[...]) + e1b[...]), e2[...]) + e2b[...]
        ang = tn_r[...] * TEMB_SCALE
        u = (dot(x_r[...], p1x[...]) + dot(agg, p1a[...])
             + jnp.cos(ang) * p1t0[...] + jnp.sin(ang) * p1t1[...]
             + p1b[...])
        h = dot(relu(u), p2[...]) + p2b[...]
        o = gam * h + bet
        out_ref[...] = o
        if has_next:
            xwn_ref[...] = dot(o, wnext[...])

    out_specs = [row(out_dim)]
    out_shape = [jax.ShapeDtypeStruct((N_NODES, out_dim), F32)]
    if has_next:
        out_specs.append(row(16))
        out_shape.append(jax.ShapeDtypeStruct((N_PAD, 16), F32))
    res = pl.pallas_call(
        body,
        grid=(N_NODES // BN,),
        in_specs=in_specs,
        out_specs=out_specs,
        out_shape=out_shape,
    )(Sp, Sp, S0, S0, x, tn, conds, *weights)
    return res if has_next else (res[0], None)


_INS = (7, 8, 8)
_OUTS = (8, 8, 5)


def _prep_layer(p, in_dim, nxt_w1x):
    (w1, b1), (w2, b2) = p['psi']
    (q1, q1b), (q2, q2b) = p['phi']
    (g1, g1b), (g2, g2b) = p['gamma']
    (e1, e1b), (e2, e2b) = p['beta']
    wl = {
        'w1e': w1[in_dim:], 'b1': b1.reshape(1, -1),
        'w2psi': w2, 'b2psi': b2.reshape(1, -1),
        'p1x': q1[:in_dim], 'p1a': q1[in_dim:in_dim + 8],
        'p1t0': q1[in_dim + 8:in_dim + 9], 'p1t1': q1[in_dim + 9:in_dim + 10],
        'p1b': q1b.reshape(1, -1), 'p2': q2, 'p2b': q2b.reshape(1, -1),
        'g1': g1, 'g1b': g1b.reshape(1, -1), 'g2': g2, 'g2b': g2b.reshape(1, -1),
        'e1': e1, 'e1b': e1b.reshape(1, -1), 'e2': e2, 'e2b': e2b.reshape(1, -1),
    }
    if nxt_w1x is not None:
        wl['wnext'] = jnp.concatenate(
            [nxt_w1x, jnp.zeros_like(nxt_w1x)], axis=1)
    return wl


def kernel(x_t, active_sites, edge_index, edge_attr, conds, time_node, params):
    layers = [params['l0'], params['l1'], params['l2']]
    w1xs = [p['psi'][0][0][:din] for p, din in zip(layers, _INS)]
    wls = [_prep_layer(p, din, w1xs[i + 1] if i < 2 else None)
           for i, (p, din) in enumerate(zip(layers, _INS))]
    for wl in wls:
        wl['wc'] = params['cond'][0]
        wl['bc'] = params['cond'][1].reshape(1, -1)

    eaT = edge_attr.T
    x0 = jnp.concatenate([x_t, active_sites], axis=1)
    tn = time_node.reshape(N_NODES, 1)
    z16 = jnp.zeros((N_PAD, 16), F32)
    w0p = jnp.concatenate([w1xs[0], jnp.zeros((_INS[0], 8), F32)], axis=1)
    sd = jnp.transpose(
        edge_index.reshape(2, 8, N_EDGES // 8), (0, 2, 1)).reshape(2, -1)

    def wblk(l, e8):
        wpad = jnp.concatenate(
            [wls[l]['w1e'],
             jnp.zeros((EA_FEATS, 8), F32)], axis=1)
        blk = jax.scipy.linalg.block_diag(*([wpad] * 8))
        brow = jnp.concatenate(
            [wls[l]['b1'], jnp.full((1, 1), e8, F32),
             jnp.zeros((1, 7), F32)], axis=1)
        return blk, jnp.tile(brow, (1, 8))

    wb0, bb0 = wblk(0, 1.0)
    wb1, bb1 = wblk(1, 0.0)
    wb2, bb2 = wblk(2, 0.0)

    pre0 = _edge_prep(eaT, wb0, bb0).reshape(-1)
    xw16 = _xw0(x0, w0p)
    Sp0 = _sc_edge(sd, pre0, xw16, z16)
    pre1 = _edge_prep(eaT, wb1, bb1).reshape(-1)
    x1, xw16 = _node(0, Sp0, Sp0, x0, tn, conds, wls[0], 8, True)
    Sp1 = _sc_edge(sd, pre1, xw16, z16)
    pre2 = _edge_prep(eaT, wb2, bb2).reshape(-1)
    x2, xw16 = _node(1, Sp1, Sp0, x1, tn, conds, wls[1], 8, True)
    Sp2 = _sc_edge(sd, pre2, xw16, z16)
    x3, _ = _node(2, Sp2, Sp0, x2, tn, conds, wls[2], 5, False)
    return x3
